# baseline (speedup 1.0000x reference)
import functools

import jax
import jax.numpy as jnp
from jax import lax
from jax.experimental import pallas as pl
from jax.experimental.pallas import tpu as pltpu
from jax.experimental.pallas import tpu_sc as plsc

_T = 4
_NC, _NS, _L = 2, 16, 16
_NW = _NC * _NS

_SELU_SCALE = 1.0507009873554805
_SELU_ALPHA = 1.6732632423543772


def _pre_body(h_ref, wa_ref, wb_ref, bm_ref, p_ref, q_ref):
    h = h_ref[...]
    p_ref[...] = jnp.dot(h, wa_ref[...], preferred_element_type=jnp.float32)
    q_ref[...] = (jnp.dot(h, wb_ref[...], preferred_element_type=jnp.float32)
                  + bm_ref[...])


def _make_pre(N, D, BR):
    G = N // BR
    return pl.pallas_call(
        _pre_body,
        grid=(G,),
        in_specs=[
            pl.BlockSpec((BR, D), lambda i: (i, 0)),
            pl.BlockSpec((D, D), lambda i: (0, 0)),
            pl.BlockSpec((D, D), lambda i: (0, 0)),
            pl.BlockSpec((1, D), lambda i: (0, 0)),
        ],
        out_specs=[
            pl.BlockSpec((BR, D), lambda i: (i, 0)),
            pl.BlockSpec((BR, D), lambda i: (i, 0)),
        ],
        out_shape=[jax.ShapeDtypeStruct((N, D), jnp.float32)] * 2,
    )


def _gru_body(parts_ref, h_ref, wk_ref, uk_ref, bx_ref, bh_ref, out_ref):
    D = h_ref.shape[1]
    x = parts_ref[0] + parts_ref[1]
    h = h_ref[...]
    xm = jnp.dot(x, wk_ref[...], preferred_element_type=jnp.float32) + bx_ref[...]
    hm = jnp.dot(h, uk_ref[...], preferred_element_type=jnp.float32) + bh_ref[...]
    z = jax.nn.sigmoid(xm[:, :D] + hm[:, :D])
    r = jax.nn.sigmoid(xm[:, D:2 * D] + hm[:, D:2 * D])
    hh = jnp.tanh(xm[:, 2 * D:] + r * hm[:, 2 * D:])
    out_ref[...] = z * h + (1.0 - z) * hh


def _make_gru(N, D, BR):
    G = N // BR
    return pl.pallas_call(
        _gru_body,
        grid=(G,),
        in_specs=[
            pl.BlockSpec((_NC, BR, D), lambda i: (0, i, 0)),
            pl.BlockSpec((BR, D), lambda i: (i, 0)),
            pl.BlockSpec((D, 3 * D), lambda i: (0, 0)),
            pl.BlockSpec((D, 3 * D), lambda i: (0, 0)),
            pl.BlockSpec((1, 3 * D), lambda i: (0, 0)),
            pl.BlockSpec((1, 3 * D), lambda i: (0, 0)),
        ],
        out_specs=pl.BlockSpec((BR, D), lambda i: (i, 0)),
        out_shape=jax.ShapeDtypeStruct((N, D), jnp.float32),
    )


def _grupre_body(parts_ref, h_ref, wk_ref, uk_ref, bx_ref, bh_ref,
                 wa_ref, wb_ref, bm_ref, out_ref, p_ref, q_ref):
    D = h_ref.shape[1]
    x = parts_ref[0] + parts_ref[1]
    h = h_ref[...]
    xm = jnp.dot(x, wk_ref[...], preferred_element_type=jnp.float32) + bx_ref[...]
    hm = jnp.dot(h, uk_ref[...], preferred_element_type=jnp.float32) + bh_ref[...]
    z = jax.nn.sigmoid(xm[:, :D] + hm[:, :D])
    r = jax.nn.sigmoid(xm[:, D:2 * D] + hm[:, D:2 * D])
    hh = jnp.tanh(xm[:, 2 * D:] + r * hm[:, 2 * D:])
    hn = z * h + (1.0 - z) * hh
    out_ref[...] = hn
    p_ref[...] = jnp.dot(hn, wa_ref[...], preferred_element_type=jnp.float32)
    q_ref[...] = (jnp.dot(hn, wb_ref[...], preferred_element_type=jnp.float32)
                  + bm_ref[...])


def _make_grupre(N, D, BR):
    G = N // BR
    full = lambda shape: pl.BlockSpec(shape, lambda i: tuple(0 for _ in shape))
    row = pl.BlockSpec((BR, D), lambda i: (i, 0))
    return pl.pallas_call(
        _grupre_body,
        grid=(G,),
        in_specs=[
            pl.BlockSpec((_NC, BR, D), lambda i: (0, i, 0)),
            row,
            full((D, 3 * D)), full((D, 3 * D)),
            full((1, 3 * D)), full((1, 3 * D)),
            full((D, D)), full((D, D)), full((1, D)),
        ],
        out_specs=[row, row, row],
        out_shape=[jax.ShapeDtypeStruct((N, D), jnp.float32)] * 3,
    )


def _gruread_body(parts_ref, h_ref, wk_ref, uk_ref, bx_ref, bh_ref,
                  w1_ref, b1_ref, w2_ref, b2_ref, w3_ref, b3_ref,
                  o_ref, acc_ref):
    D = h_ref.shape[1]
    i = pl.program_id(0)
    x = parts_ref[0] + parts_ref[1]
    h = h_ref[...]
    xm = jnp.dot(x, wk_ref[...], preferred_element_type=jnp.float32) + bx_ref[...]
    hm = jnp.dot(h, uk_ref[...], preferred_element_type=jnp.float32) + bh_ref[...]
    z = jax.nn.sigmoid(xm[:, :D] + hm[:, :D])
    r = jax.nn.sigmoid(xm[:, D:2 * D] + hm[:, D:2 * D])
    hh = jnp.tanh(xm[:, 2 * D:] + r * hm[:, 2 * D:])
    hn = z * h + (1.0 - z) * hh
    part = jnp.sum(hn, axis=0, keepdims=True)

    @pl.when(i == 0)
    def _init():
        acc_ref[...] = part

    @pl.when(i > 0)
    def _acc():
        acc_ref[...] += part

    @pl.when(i == pl.num_programs(0) - 1)
    def _readout():
        s = acc_ref[...]
        y = _tc_selu(jnp.dot(s, w1_ref[...], preferred_element_type=jnp.float32)
                     + b1_ref[...])
        y = _tc_selu(jnp.dot(y, w2_ref[...], preferred_element_type=jnp.float32)
                     + b2_ref[...])
        o_ref[...] = (jnp.dot(y, w3_ref[...], preferred_element_type=jnp.float32)
                      + b3_ref[...])


def _make_gruread(N, D, RU, BR):
    G = N // BR
    full = lambda shape: pl.BlockSpec(shape, lambda i: tuple(0 for _ in shape))
    return pl.pallas_call(
        _gruread_body,
        grid=(G,),
        in_specs=[
            pl.BlockSpec((_NC, BR, D), lambda i: (0, i, 0)),
            pl.BlockSpec((BR, D), lambda i: (i, 0)),
            full((D, 3 * D)), full((D, 3 * D)),
            full((1, 3 * D)), full((1, 3 * D)),
            full((D, RU)), full((1, RU)),
            full((RU, RU)), full((1, RU)),
            full((RU, 1)), full((1, 1)),
        ],
        out_specs=full((1, 1)),
        out_shape=jax.ShapeDtypeStruct((1, 1), jnp.float32),
        scratch_shapes=[pltpu.VMEM((1, D), jnp.float32)],
    )


def _tc_selu(x):
    return _SELU_SCALE * jnp.where(x > 0, x, _SELU_ALPHA * (jnp.exp(x) - 1.0))


def _readout_body(h_ref, w1_ref, b1_ref, w2_ref, b2_ref, w3_ref, b3_ref, o_ref):
    ssum = jnp.sum(h_ref[...], axis=0, keepdims=True)
    x = _tc_selu(jnp.dot(ssum, w1_ref[...], preferred_element_type=jnp.float32)
                 + b1_ref[...])
    x = _tc_selu(jnp.dot(x, w2_ref[...], preferred_element_type=jnp.float32)
                 + b2_ref[...])
    o_ref[...] = (jnp.dot(x, w3_ref[...], preferred_element_type=jnp.float32)
                  + b3_ref[...])


def _make_readout(N, D, RU):
    return pl.pallas_call(
        _readout_body,
        out_shape=jax.ShapeDtypeStruct((1, 1), jnp.float32),
    )


def _make_edge(N, E, D, CH, NB=2):
    EPW = E // _NW
    NCH = EPW // CH
    assert EPW % CH == 0 and NCH % NB == 0
    RPS = (N // _NS) // 8 * 8
    REM = N - RPS * _NS
    mesh = plsc.VectorSubcoreMesh(core_axis_name="c", subcore_axis_name="s")
    al = _SELU_ALPHA

    NI = 8
    assert NCH % NI == 0 and NCH > NI

    @functools.partial(
        pl.kernel,
        out_type=jax.ShapeDtypeStruct((_NC, N, D), jnp.float32),
        mesh=mesh,
        scratch_types=[
            pltpu.VMEM((NI, CH), jnp.int32),
            pltpu.VMEM((NI, CH), jnp.int32),
            pltpu.VMEM((NB, CH, D), jnp.float32),
            pltpu.VMEM((NB, CH, D), jnp.float32),
            pltpu.VMEM((NB, CH, D), jnp.float32),
            pltpu.VMEM_SHARED((N, D), jnp.float32),
            [pltpu.SemaphoreType.DMA] * NB,
            [pltpu.SemaphoreType.DMA] * NB,
            [pltpu.SemaphoreType.DMA] * NI,
        ],
    )
    def edge(p_hbm, q_hbm, f_hbm, s_hbm, z_hbm, out_hbm,
             fbuf, sbuf, bufp, bufq, bufo, agg, sems, osems, isems):
        cid = lax.axis_index("c")
        sid = lax.axis_index("s")
        wid = sid * _NC + cid

        def issue_idx(k, bi):
            pltpu.async_copy(f_hbm.at[wid, k], fbuf.at[bi], isems[bi])
            pltpu.async_copy(s_hbm.at[wid, k], sbuf.at[bi], isems[bi])

        def wait_idx(k, bi):
            pltpu.make_async_copy(f_hbm.at[wid, k], fbuf.at[bi], isems[bi]).wait()
            pltpu.make_async_copy(s_hbm.at[wid, k], sbuf.at[bi], isems[bi]).wait()

        def issue(b, bi):
            pltpu.async_copy(p_hbm.at[fbuf.at[bi]], bufp.at[b], sems[b])
            pltpu.async_copy(q_hbm.at[sbuf.at[bi]], bufq.at[b], sems[b])

        def wait(b, bi):
            pltpu.make_async_copy(p_hbm.at[fbuf.at[bi]], bufp.at[b], sems[b]).wait()
            pltpu.make_async_copy(q_hbm.at[sbuf.at[bi]], bufq.at[b], sems[b]).wait()

        def wait_scatter(b):
            pltpu.make_async_copy(bufo.at[b], agg.at[sbuf.at[0]], osems[b]).wait()

        for k in range(NI - NB):
            issue_idx(k, k)
        r0 = sid * RPS
        pltpu.sync_copy(z_hbm.at[pl.ds(r0, RPS)], agg.at[pl.ds(r0, RPS)])
        if REM:
            @pl.when(sid == _NS - 1)
            def _zero_rem():
                rr = RPS * _NS
                pltpu.sync_copy(z_hbm.at[pl.ds(rr, REM)], agg.at[pl.ds(rr, REM)])
        plsc.subcore_barrier()
        for b in range(NB):
            wait_idx(b, b)
            issue(b, b)

        def group(g, carry):
            k0 = g * NI
            for u in range(NI):
                k = k0 + u
                b = u % NB
                bi = u
                wait(b, bi)

                @pl.when(k >= NB)
                def _drain_scatter():
                    wait_scatter(b)

                @pl.when(k + NI - NB < NCH)
                def _next_idx():
                    issue_idx(k + NI - NB, (u + NI - NB) % NI)

                def row(i2, c2):
                    for jj in range(5):
                        i = i2 * 5 + jj
                        for j in range(D // _L):
                            sl = pl.ds(j * _L, _L)
                            x = bufp[b, i, sl] + bufq[b, i, sl]
                            e = al * jnp.exp(x) - al
                            bufo[b, i, sl] = jnp.where(x > 0.0, x, e)
                    return c2

                lax.fori_loop(0, CH // 5, row, 0)

                pltpu.async_copy(bufo.at[b], agg.at[sbuf.at[bi]], osems[b],
                                 add=True)

                @pl.when(k + NB < NCH)
                def _next_gather():
                    wait_idx(k + NB, (u + NB) % NI)
                    issue(b, (u + NB) % NI)
            return carry

        lax.fori_loop(0, NCH // NI, group, 0)
        for b in range(NB):
            wait_scatter(b)
        plsc.subcore_barrier()
        pltpu.sync_copy(agg.at[pl.ds(r0, RPS)],
                        out_hbm.at[cid, pl.ds(r0, RPS)])
        if REM:
            @pl.when(sid == _NS - 1)
            def _out_rem():
                rr = RPS * _NS
                pltpu.sync_copy(agg.at[pl.ds(rr, REM)],
                                out_hbm.at[cid, pl.ds(rr, REM)])

    return edge


def kernel(link_state, first_critic, second_critic, num_edges_critic,
           Wm, bm, Wk, Uk, b_gru, W1, b1, W2, b2, W3, b3):
    N, D = link_state.shape
    E = first_critic.shape[0]
    RU = W2.shape[0]
    CH = 50
    f = first_critic.astype(jnp.int32).reshape(_NW, -1, CH)
    s = second_critic.astype(jnp.int32).reshape(_NW, -1, CH)
    wa = Wm[:D]
    wb = Wm[D:]
    wks = Wk * _SELU_SCALE
    bm2 = bm.reshape(1, D)
    bx = b_gru[0].reshape(1, 3 * D)
    bh = b_gru[1].reshape(1, 3 * D)
    zeros = jnp.zeros((N, D), jnp.float32)

    pre = _make_pre(N, D, 2000)
    grupre = _make_grupre(N, D, 2000)
    gruread = _make_gruread(N, D, RU, 2000)
    edge = _make_edge(N, E, D, CH)

    h = link_state
    p, q = pre(h, wa, wb, bm2)
    for _ in range(_T - 1):
        parts = edge(p, q, f, s, zeros)
        h, p, q = grupre(parts, h, wks, Uk, bx, bh, wa, wb, bm2)
    parts = edge(p, q, f, s, zeros)
    return gruread(parts, h, wks, Uk, bx, bh,
                   W1, b1.reshape(1, RU), W2, b2.reshape(1, RU),
                   W3, b3.reshape(1, 1))


# speedup vs baseline: 1.4640x; 1.4640x over previous
import functools

import jax
import jax.numpy as jnp
from jax import lax
from jax.experimental import pallas as pl
from jax.experimental.pallas import tpu as pltpu
from jax.experimental.pallas import tpu_sc as plsc

_T = 4
_NC, _NS, _L = 2, 16, 16
_NW = _NC * _NS

_SELU_SCALE = 1.0507009873554805
_SELU_ALPHA = 1.6732632423543772


def _pre_body(h_ref, wa_ref, wb_ref, bm_ref, p_ref, q_ref):
    h = h_ref[...]
    p_ref[...] = jnp.dot(h, wa_ref[...], preferred_element_type=jnp.float32)
    q_ref[...] = (jnp.dot(h, wb_ref[...], preferred_element_type=jnp.float32)
                  + bm_ref[...])


def _make_pre(N, D, BR):
    G = N // BR
    return pl.pallas_call(
        _pre_body,
        grid=(G,),
        in_specs=[
            pl.BlockSpec((BR, D), lambda i: (i, 0)),
            pl.BlockSpec((D, D), lambda i: (0, 0)),
            pl.BlockSpec((D, D), lambda i: (0, 0)),
            pl.BlockSpec((1, D), lambda i: (0, 0)),
        ],
        out_specs=[
            pl.BlockSpec((BR, D), lambda i: (i, 0)),
            pl.BlockSpec((BR, D), lambda i: (i, 0)),
        ],
        out_shape=[jax.ShapeDtypeStruct((N, D), jnp.float32)] * 2,
    )


def _gru_body(parts_ref, h_ref, wk_ref, uk_ref, bx_ref, bh_ref, out_ref):
    D = h_ref.shape[1]
    x = parts_ref[0] + parts_ref[1]
    h = h_ref[...]
    xm = jnp.dot(x, wk_ref[...], preferred_element_type=jnp.float32) + bx_ref[...]
    hm = jnp.dot(h, uk_ref[...], preferred_element_type=jnp.float32) + bh_ref[...]
    z = jax.nn.sigmoid(xm[:, :D] + hm[:, :D])
    r = jax.nn.sigmoid(xm[:, D:2 * D] + hm[:, D:2 * D])
    hh = jnp.tanh(xm[:, 2 * D:] + r * hm[:, 2 * D:])
    out_ref[...] = z * h + (1.0 - z) * hh


def _make_gru(N, D, BR):
    G = N // BR
    return pl.pallas_call(
        _gru_body,
        grid=(G,),
        in_specs=[
            pl.BlockSpec((_NC, BR, D), lambda i: (0, i, 0)),
            pl.BlockSpec((BR, D), lambda i: (i, 0)),
            pl.BlockSpec((D, 3 * D), lambda i: (0, 0)),
            pl.BlockSpec((D, 3 * D), lambda i: (0, 0)),
            pl.BlockSpec((1, 3 * D), lambda i: (0, 0)),
            pl.BlockSpec((1, 3 * D), lambda i: (0, 0)),
        ],
        out_specs=pl.BlockSpec((BR, D), lambda i: (i, 0)),
        out_shape=jax.ShapeDtypeStruct((N, D), jnp.float32),
    )


def _grupre_body(parts_ref, h_ref, wk_ref, uk_ref, bx_ref, bh_ref,
                 wa_ref, wb_ref, bm_ref, out_ref, p_ref, q_ref):
    D = h_ref.shape[1]
    x = parts_ref[0] + parts_ref[1]
    h = h_ref[...]
    xm = jnp.dot(x, wk_ref[...], preferred_element_type=jnp.float32) + bx_ref[...]
    hm = jnp.dot(h, uk_ref[...], preferred_element_type=jnp.float32) + bh_ref[...]
    z = jax.nn.sigmoid(xm[:, :D] + hm[:, :D])
    r = jax.nn.sigmoid(xm[:, D:2 * D] + hm[:, D:2 * D])
    hh = jnp.tanh(xm[:, 2 * D:] + r * hm[:, 2 * D:])
    hn = z * h + (1.0 - z) * hh
    out_ref[...] = hn
    p_ref[...] = jnp.dot(hn, wa_ref[...], preferred_element_type=jnp.float32)
    q_ref[...] = (jnp.dot(hn, wb_ref[...], preferred_element_type=jnp.float32)
                  + bm_ref[...])


def _make_grupre(N, D, BR):
    G = N // BR
    full = lambda shape: pl.BlockSpec(shape, lambda i: tuple(0 for _ in shape))
    row = pl.BlockSpec((BR, D), lambda i: (i, 0))
    return pl.pallas_call(
        _grupre_body,
        grid=(G,),
        in_specs=[
            pl.BlockSpec((_NC, BR, D), lambda i: (0, i, 0)),
            row,
            full((D, 3 * D)), full((D, 3 * D)),
            full((1, 3 * D)), full((1, 3 * D)),
            full((D, D)), full((D, D)), full((1, D)),
        ],
        out_specs=[row, row, row],
        out_shape=[jax.ShapeDtypeStruct((N, D), jnp.float32)] * 3,
    )


def _gruread_body(parts_ref, h_ref, wk_ref, uk_ref, bx_ref, bh_ref,
                  w1_ref, b1_ref, w2_ref, b2_ref, w3_ref, b3_ref,
                  o_ref, acc_ref):
    D = h_ref.shape[1]
    i = pl.program_id(0)
    x = parts_ref[0] + parts_ref[1]
    h = h_ref[...]
    xm = jnp.dot(x, wk_ref[...], preferred_element_type=jnp.float32) + bx_ref[...]
    hm = jnp.dot(h, uk_ref[...], preferred_element_type=jnp.float32) + bh_ref[...]
    z = jax.nn.sigmoid(xm[:, :D] + hm[:, :D])
    r = jax.nn.sigmoid(xm[:, D:2 * D] + hm[:, D:2 * D])
    hh = jnp.tanh(xm[:, 2 * D:] + r * hm[:, 2 * D:])
    hn = z * h + (1.0 - z) * hh
    part = jnp.sum(hn, axis=0, keepdims=True)

    @pl.when(i == 0)
    def _init():
        acc_ref[...] = part

    @pl.when(i > 0)
    def _acc():
        acc_ref[...] += part

    @pl.when(i == pl.num_programs(0) - 1)
    def _readout():
        s = acc_ref[...]
        y = _tc_selu(jnp.dot(s, w1_ref[...], preferred_element_type=jnp.float32)
                     + b1_ref[...])
        y = _tc_selu(jnp.dot(y, w2_ref[...], preferred_element_type=jnp.float32)
                     + b2_ref[...])
        o_ref[...] = (jnp.dot(y, w3_ref[...], preferred_element_type=jnp.float32)
                      + b3_ref[...])


def _make_gruread(N, D, RU, BR):
    G = N // BR
    full = lambda shape: pl.BlockSpec(shape, lambda i: tuple(0 for _ in shape))
    return pl.pallas_call(
        _gruread_body,
        grid=(G,),
        in_specs=[
            pl.BlockSpec((_NC, BR, D), lambda i: (0, i, 0)),
            pl.BlockSpec((BR, D), lambda i: (i, 0)),
            full((D, 3 * D)), full((D, 3 * D)),
            full((1, 3 * D)), full((1, 3 * D)),
            full((D, RU)), full((1, RU)),
            full((RU, RU)), full((1, RU)),
            full((RU, 1)), full((1, 1)),
        ],
        out_specs=full((1, 1)),
        out_shape=jax.ShapeDtypeStruct((1, 1), jnp.float32),
        scratch_shapes=[pltpu.VMEM((1, D), jnp.float32)],
    )


def _tc_selu(x):
    return _SELU_SCALE * jnp.where(x > 0, x, _SELU_ALPHA * (jnp.exp(x) - 1.0))


def _readout_body(h_ref, w1_ref, b1_ref, w2_ref, b2_ref, w3_ref, b3_ref, o_ref):
    ssum = jnp.sum(h_ref[...], axis=0, keepdims=True)
    x = _tc_selu(jnp.dot(ssum, w1_ref[...], preferred_element_type=jnp.float32)
                 + b1_ref[...])
    x = _tc_selu(jnp.dot(x, w2_ref[...], preferred_element_type=jnp.float32)
                 + b2_ref[...])
    o_ref[...] = (jnp.dot(x, w3_ref[...], preferred_element_type=jnp.float32)
                  + b3_ref[...])


def _make_readout(N, D, RU):
    return pl.pallas_call(
        _readout_body,
        out_shape=jax.ShapeDtypeStruct((1, 1), jnp.float32),
    )


def _make_edge(N, E, D, CH, NB=2):
    EPW = E // _NW
    NCH = EPW // CH
    assert EPW % CH == 0 and NCH % NB == 0
    RPS = (N // _NS) // 8 * 8
    REM = N - RPS * _NS
    mesh = plsc.VectorSubcoreMesh(core_axis_name="c", subcore_axis_name="s")
    al = _SELU_ALPHA

    NI = 8
    assert NCH % NI == 0 and NCH > NI

    @functools.partial(
        pl.kernel,
        out_type=jax.ShapeDtypeStruct((_NC, N, D), jnp.float32),
        mesh=mesh,
        scratch_types=[
            pltpu.VMEM((NI, CH), jnp.int32),
            pltpu.VMEM((NI, CH), jnp.int32),
            pltpu.VMEM((NB, CH, D), jnp.float32),
            pltpu.VMEM((NB, CH, D), jnp.float32),
            pltpu.VMEM((NB, CH, D), jnp.float32),
            pltpu.VMEM_SHARED((N, D), jnp.float32),
            [pltpu.SemaphoreType.DMA] * NB,
            [pltpu.SemaphoreType.DMA] * NB,
            [pltpu.SemaphoreType.DMA] * NI,
        ],
    )
    def edge(p_hbm, q_hbm, f_hbm, s_hbm, z_hbm, out_hbm,
             fbuf, sbuf, bufp, bufq, bufo, agg, sems, osems, isems):
        cid = lax.axis_index("c")
        sid = lax.axis_index("s")
        wid = sid * _NC + cid

        def issue_idx(k, bi):
            pltpu.async_copy(f_hbm.at[wid, k], fbuf.at[bi], isems[bi])
            pltpu.async_copy(s_hbm.at[wid, k], sbuf.at[bi], isems[bi])

        def wait_idx(k, bi):
            pltpu.make_async_copy(f_hbm.at[wid, k], fbuf.at[bi], isems[bi]).wait()
            pltpu.make_async_copy(s_hbm.at[wid, k], sbuf.at[bi], isems[bi]).wait()

        def issue(b, bi):
            pltpu.async_copy(p_hbm.at[fbuf.at[bi]], bufp.at[b], sems[b])
            pltpu.async_copy(q_hbm.at[sbuf.at[bi]], bufq.at[b], sems[b])

        def wait(b, bi):
            pltpu.make_async_copy(p_hbm.at[fbuf.at[bi]], bufp.at[b], sems[b]).wait()
            pltpu.make_async_copy(q_hbm.at[sbuf.at[bi]], bufq.at[b], sems[b]).wait()

        def wait_scatter(b):
            pltpu.make_async_copy(bufo.at[b], agg.at[sbuf.at[0]], osems[b]).wait()

        for k in range(NI - NB):
            issue_idx(k, k)
        r0 = sid * RPS
        pltpu.sync_copy(z_hbm.at[pl.ds(r0, RPS)], agg.at[pl.ds(r0, RPS)])
        if REM:
            @pl.when(sid == _NS - 1)
            def _zero_rem():
                rr = RPS * _NS
                pltpu.sync_copy(z_hbm.at[pl.ds(rr, REM)], agg.at[pl.ds(rr, REM)])
        plsc.subcore_barrier()
        for b in range(NB):
            wait_idx(b, b)
            issue(b, b)

        def group(g, carry):
            k0 = g * NI
            for u in range(NI):
                k = k0 + u
                b = u % NB
                bi = u
                wait(b, bi)

                @pl.when(k >= NB)
                def _drain_scatter():
                    wait_scatter(b)

                @pl.when(k + NI - NB < NCH)
                def _next_idx():
                    issue_idx(k + NI - NB, (u + NI - NB) % NI)

                def row(i2, c2):
                    for jj in range(2):
                        i = i2 * 2 + jj
                        for j in range(D // _L):
                            sl = pl.ds(j * _L, _L)
                            x = bufp[b, i, sl] + bufq[b, i, sl]
                            e = al * jnp.exp(x) - al
                            bufo[b, i, sl] = jnp.where(x > 0.0, x, e)
                    return c2

                lax.fori_loop(0, CH // 2, row, 0)

                pltpu.async_copy(bufo.at[b], agg.at[sbuf.at[bi]], osems[b],
                                 add=True)

                @pl.when(k + NB < NCH)
                def _next_gather():
                    wait_idx(k + NB, (u + NB) % NI)
                    issue(b, (u + NB) % NI)
            return carry

        lax.fori_loop(0, NCH // NI, group, 0)
        for b in range(NB):
            wait_scatter(b)
        plsc.subcore_barrier()
        pltpu.sync_copy(agg.at[pl.ds(r0, RPS)],
                        out_hbm.at[cid, pl.ds(r0, RPS)])
        if REM:
            @pl.when(sid == _NS - 1)
            def _out_rem():
                rr = RPS * _NS
                pltpu.sync_copy(agg.at[pl.ds(rr, REM)],
                                out_hbm.at[cid, pl.ds(rr, REM)])

    return edge


def kernel(link_state, first_critic, second_critic, num_edges_critic,
           Wm, bm, Wk, Uk, b_gru, W1, b1, W2, b2, W3, b3):
    N, D = link_state.shape
    E = first_critic.shape[0]
    RU = W2.shape[0]
    CH = 50
    f = first_critic.astype(jnp.int32).reshape(_NW, -1, CH)
    s = second_critic.astype(jnp.int32).reshape(_NW, -1, CH)
    wa = Wm[:D]
    wb = Wm[D:]
    wks = Wk * _SELU_SCALE
    bm2 = bm.reshape(1, D)
    bx = b_gru[0].reshape(1, 3 * D)
    bh = b_gru[1].reshape(1, 3 * D)
    zeros = jnp.zeros((N, D), jnp.float32)

    pre = _make_pre(N, D, 2000)
    grupre = _make_grupre(N, D, 2000)
    gruread = _make_gruread(N, D, RU, 2000)
    edge = _make_edge(N, E, D, CH)

    h = link_state
    p, q = pre(h, wa, wb, bm2)
    for _ in range(_T - 1):
        parts = edge(p, q, f, s, zeros)
        h, p, q = grupre(parts, h, wks, Uk, bx, bh, wa, wb, bm2)
    parts = edge(p, q, f, s, zeros)
    return gruread(parts, h, wks, Uk, bx, bh,
                   W1, b1.reshape(1, RU), W2, b2.reshape(1, RU),
                   W3, b3.reshape(1, 1))
